# baseline (speedup 1.0000x reference)
import jax
import jax.numpy as jnp
from jax.experimental import pallas as pl
from jax.experimental.pallas import tpu as pltpu


def _gather_kernel(idx_ref, lut_ref, out_ref, warm_ref, buf, row_sem, out_sem):
    B = out_ref.shape[0]
    pltpu.make_async_copy(
        lut_ref.at[pl.ds(0, 1), :], buf.at[pl.ds(B, 1), :], row_sem
    ).start()
    pltpu.make_async_copy(
        buf.at[pl.ds(B, 1), :], warm_ref.at[pl.ds(0, 1), :], out_sem
    ).start()
    for b in range(B):
        pltpu.make_async_copy(
            lut_ref.at[pl.ds(idx_ref[b], 1), :],
            buf.at[pl.ds(b, 1), :],
            row_sem,
        ).start()
    pltpu.make_async_copy(
        lut_ref.at[pl.ds(0, 1), :], buf.at[pl.ds(B, 1), :], row_sem
    ).wait()
    pltpu.make_async_copy(
        lut_ref.at[pl.ds(0, B), :], buf.at[pl.ds(0, B), :], row_sem
    ).wait()
    pltpu.make_async_copy(
        buf.at[pl.ds(0, B), :], out_ref.at[pl.ds(0, B), :], out_sem
    ).start()
    pltpu.make_async_copy(
        buf.at[pl.ds(B, 1), :], warm_ref.at[pl.ds(0, 1), :], out_sem
    ).wait()
    pltpu.make_async_copy(
        buf.at[pl.ds(0, B), :], out_ref.at[pl.ds(0, B), :], out_sem
    ).wait()


def kernel(task_indices, fused_table):
    B = task_indices.shape[0]
    D = fused_table.shape[1]
    out, _ = pl.pallas_call(
        _gather_kernel,
        out_shape=(
            jax.ShapeDtypeStruct((B, D), fused_table.dtype),
            jax.ShapeDtypeStruct((1, D), fused_table.dtype),
        ),
        in_specs=[
            pl.BlockSpec(memory_space=pltpu.MemorySpace.SMEM),
            pl.BlockSpec(memory_space=pl.ANY),
        ],
        out_specs=(
            pl.BlockSpec(memory_space=pl.ANY),
            pl.BlockSpec(memory_space=pl.ANY),
        ),
        scratch_shapes=[
            pltpu.VMEM((B + 1, D), jnp.float32),
            pltpu.SemaphoreType.DMA,
            pltpu.SemaphoreType.DMA,
        ],
    )(task_indices.astype(jnp.int32), fused_table)
    return out


# speedup vs baseline: 1.0132x; 1.0132x over previous
import jax
import jax.numpy as jnp
from jax.experimental import pallas as pl
from jax.experimental.pallas import tpu as pltpu


def _gather_direct_kernel(idx_ref, lut_ref, out_ref, sem):
    B = out_ref.shape[0]
    for b in range(B):
        pltpu.make_async_copy(
            lut_ref.at[pl.ds(idx_ref[b], 1), :],
            out_ref.at[pl.ds(b, 1), :],
            sem,
        ).start()
    pltpu.make_async_copy(
        lut_ref.at[pl.ds(0, B), :],
        out_ref.at[pl.ds(0, B), :],
        sem,
    ).wait()


def kernel(task_indices, fused_table):
    B = task_indices.shape[0]
    return pl.pallas_call(
        _gather_direct_kernel,
        out_shape=jax.ShapeDtypeStruct((B, fused_table.shape[1]), fused_table.dtype),
        in_specs=[
            pl.BlockSpec(memory_space=pltpu.MemorySpace.SMEM),
            pl.BlockSpec(memory_space=pl.ANY),
        ],
        out_specs=pl.BlockSpec(memory_space=pltpu.MemorySpace.VMEM),
        scratch_shapes=[pltpu.SemaphoreType.DMA],
    )(task_indices.astype(jnp.int32), fused_table)


# speedup vs baseline: 1.0231x; 1.0098x over previous
import jax
import jax.numpy as jnp
from jax.experimental import pallas as pl
from jax.experimental.pallas import tpu as pltpu


def _gather_direct_kernel(idx_ref, lut_ref, out_ref, sem):
    B = out_ref.shape[0]
    for b in range(B):
        pltpu.make_async_copy(
            lut_ref.at[pl.ds(idx_ref[b], 1), :],
            out_ref.at[pl.ds(b, 1), :],
            sem,
        ).start()
    pltpu.make_async_copy(
        lut_ref.at[pl.ds(0, B), :],
        out_ref.at[pl.ds(0, B), :],
        sem,
    ).wait()


def kernel(task_indices, fused_table):
    B = task_indices.shape[0]
    return pl.pallas_call(
        _gather_direct_kernel,
        out_shape=jax.ShapeDtypeStruct((B, fused_table.shape[1]), fused_table.dtype),
        in_specs=[
            pl.BlockSpec(memory_space=pltpu.MemorySpace.SMEM),
            pl.BlockSpec(memory_space=pl.ANY),
        ],
        out_specs=pl.BlockSpec(memory_space=pltpu.MemorySpace.VMEM),
        scratch_shapes=[pltpu.SemaphoreType.DMA],
        compiler_params=pltpu.CompilerParams(
            disable_bounds_checks=True,
            disable_semaphore_checks=True,
            skip_device_barrier=True,
        ),
    )(task_indices.astype(jnp.int32), fused_table)
